# baseline (speedup 1.0000x reference)
import functools

import jax
import jax.numpy as jnp
from jax import lax
from jax.experimental import pallas as pl
from jax.experimental.pallas import tpu as pltpu
from jax.experimental.pallas import tpu_sc as plsc

NC, NS, L = 2, 16, 16
NW = NC * NS
GROUP = 128
RT = 640

_f32 = jnp.float32
_i32 = jnp.int32


def _full16(v):
    return jnp.full((L,), v, _i32)


def _zeros16():
    return jnp.zeros((L,), _f32)


def _unpack16(pkv, srcv, dstv, j, g):
    for k in range(g // L):
        sl = pl.ds(k * L, L)
        v = pkv[j, sl]
        if srcv is not None:
            srcv[j, sl] = v & jnp.int32(0xFFFF)
        dstv[j, sl] = v >> 16


def _deg_body(pk2, ew2, degp, pkv, dstv, ewv, zbuf, acc, *, rpw, g):
    c = lax.axis_index("c")
    s = lax.axis_index("s")
    wid = s * NC + c

    def _zb(i, _):
        zbuf[pl.ds(i * L, L)] = _zeros16()
        return 0
    lax.fori_loop(0, RT // L, _zb, 0)
    pltpu.sync_copy(zbuf, acc.at[pl.ds(s * RT, RT)])
    plsc.subcore_barrier()

    pltpu.sync_copy(pk2.at[pl.ds(wid * rpw, rpw)], pkv)
    pltpu.sync_copy(ew2.at[pl.ds(wid * rpw, rpw)], ewv)

    def _row(j, _):
        _unpack16(pkv, None, dstv, j, g)
        pltpu.sync_copy(ewv.at[j], acc.at[dstv.at[j]], add=True)
        return 0
    lax.fori_loop(0, rpw, _row, 0)

    plsc.subcore_barrier()
    pltpu.sync_copy(acc.at[pl.ds(s * RT, RT)], degp.at[c, pl.ds(s * RT, RT)])


NBUF = 5


def _spmv_body(ht, pk2, ewf, out, *args, rpw, d, g):
    pkv, srcv, dstv, ewv = args[0], args[1], args[2], args[3]
    rows = args[4:4 + NBUF]
    gsems = args[4 + NBUF]
    ssems = args[5 + NBUF]
    acc = args[6 + NBUF]
    c = lax.axis_index("c")
    s = lax.axis_index("s")
    wid = s * NC + c

    pltpu.async_copy(pk2.at[pl.ds(wid * rpw, rpw)], pkv, gsems.at[0])
    pltpu.async_copy(ewf.at[pl.ds(wid * rpw * GROUP, rpw * GROUP)],
                     ewv.at[pl.ds(0, rpw * GROUP)], gsems.at[2])

    def _zr(e, _):
        for k in range(d // L):
            rows[0][e, pl.ds(k * L, L)] = _zeros16()
        return 0
    lax.fori_loop(0, rows[0].shape[0], _zr, 0)
    for k in range(RT // rows[0].shape[0]):
        pltpu.sync_copy(rows[0], acc.at[pl.ds(s * RT + k * rows[0].shape[0],
                                              rows[0].shape[0])])

    pltpu.make_async_copy(pk2.at[pl.ds(wid * rpw, rpw)], pkv,
                          gsems.at[0]).wait()
    pltpu.make_async_copy(ewf.at[pl.ds(wid * rpw * GROUP, rpw * GROUP)],
                          ewv.at[pl.ds(0, rpw * GROUP)], gsems.at[2]).wait()

    plsc.subcore_barrier()

    def _gather(j, b):
        pltpu.async_copy(ht.at[srcv.at[j]], rows[b].at[pl.ds(0, GROUP)],
                         gsems.at[b])

    def _gwait(j, b):
        pltpu.make_async_copy(ht.at[srcv.at[j]], rows[b].at[pl.ds(0, GROUP)],
                              gsems.at[b]).wait()

    def _mul(j, b):
        base = j * GROUP
        buf = rows[b]

        def mbody(gi):
            e0 = gi * L
            w16 = ewv[pl.ds(base + e0, L)]
            for t in range(L):
                w = w16[jnp.full((L,), t, _i32)]
                for k in range(d // L):
                    sl = pl.ds(k * L, L)
                    buf[e0 + t, sl] = buf[e0 + t, sl] * w
        plsc.parallel_loop(0, GROUP // L, 1, unroll=4)(mbody)

    for b in range(NBUF):
        _unpack16(pkv, srcv, dstv, b, g)
        _gather(b, b)

    def _quad(i, _):
        j = i * NBUF
        descs = []
        for b in range(NBUF):
            _gwait(j + b, b)
            _mul(j + b, b)
            descs.append(pltpu.async_copy(rows[b].at[pl.ds(0, GROUP)],
                                          acc.at[dstv.at[j + b]], ssems.at[b],
                                          add=True))
        for b in range(NBUF):
            descs[b].wait()

            @pl.when(j + b + NBUF < rpw)
            def _():
                _unpack16(pkv, srcv, dstv, j + b + NBUF, g)
                _gather(j + b + NBUF, b)
        return 0
    lax.fori_loop(0, rpw // NBUF, _quad, 0)

    plsc.subcore_barrier()
    pltpu.sync_copy(acc.at[pl.ds(s * RT, RT)], out.at[c, pl.ds(s * RT, RT), :])


def _deg_call(pk2, ew2):
    rows2, g = pk2.shape
    rpw = rows2 // NW
    mesh = plsc.VectorSubcoreMesh(core_axis_name="c", subcore_axis_name="s")
    body = functools.partial(_deg_body, rpw=rpw, g=g)
    return pl.kernel(
        body,
        out_type=jax.ShapeDtypeStruct((NC, NS * RT), _f32),
        mesh=mesh,
        scratch_types=[
            pltpu.VMEM((rpw, g), _i32),
            pltpu.VMEM((rpw, g), _i32),
            pltpu.VMEM((rpw, g), _f32),
            pltpu.VMEM((RT,), _f32),
            pltpu.VMEM_SHARED((NS * RT,), _f32),
        ],
    )(pk2, ew2)


def _spmv_call(ht, pk2, ewf):
    d = ht.shape[1]
    rows2, g = pk2.shape
    rpw = rows2 // NW
    mesh = plsc.VectorSubcoreMesh(core_axis_name="c", subcore_axis_name="s")
    body = functools.partial(_spmv_body, rpw=rpw, d=d, g=g)
    return pl.kernel(
        body,
        out_type=jax.ShapeDtypeStruct((NC, NS * RT, d), _f32),
        mesh=mesh,
        compiler_params=pltpu.CompilerParams(use_tc_tiling_on_sc=False),
        scratch_types=[
            pltpu.VMEM((rpw, g), _i32),
            pltpu.VMEM((rpw, g), _i32),
            pltpu.VMEM((rpw, g), _i32),
            pltpu.VMEM((rpw * g + L,), _f32),
        ] + [pltpu.VMEM((GROUP, d), _f32) for _ in range(NBUF)] + [
            pltpu.SemaphoreType.DMA((NBUF,)),
            pltpu.SemaphoreType.DMA((NBUF,)),
            pltpu.VMEM_SHARED((NS * RT, d), _f32),
        ],
    )(ht, pk2, ewf)


def _disp128(disp_ref, m, dh):
    de = disp_ref[0:m, 0:1]
    do = disp_ref[0:m, 1:2]
    return jnp.concatenate([jnp.broadcast_to(de, (m, dh)),
                            jnp.broadcast_to(do, (m, dh))], axis=1)


def _tc1_body(disp_ref, x_ref, w_ref, ht_ref, *, dh):
    m = x_ref.shape[0]
    d128 = _disp128(disp_ref, m, dh)
    ht_ref[...] = jnp.dot(x_ref[...], w_ref[...],
                          preferred_element_type=_f32) * d128


def _tc2_body(s_ref, ht_ref, disp_ref, b_ref, w_ref, h_ref, htn_ref, *, dh):
    m = ht_ref.shape[0]
    d128 = _disp128(disp_ref, m, dh)
    S = (s_ref[0] + s_ref[1])[:m]
    h = d128 * (S + ht_ref[...]) + b_ref[...][None, :]
    h_ref[...] = h
    htn_ref[...] = jnp.dot(h, w_ref[...], preferred_element_type=_f32) * d128


def _tc3_body(s_ref, ht_ref, disp_ref, b_ref, h1_ref, wa_ref, wb_ref,
              htn_ref, *, dh):
    m = ht_ref.shape[0]
    d128 = _disp128(disp_ref, m, dh)
    S = (s_ref[0] + s_ref[1])[:m]
    h2 = d128 * (S + ht_ref[...]) + b_ref[...][None, :]
    acc = (jnp.dot(h1_ref[...], wa_ref[...], preferred_element_type=_f32)
           + jnp.dot(h2, wb_ref[...], preferred_element_type=_f32))
    htn_ref[...] = acc * d128


def _tc4_body(s_ref, ht_ref, disp_ref, b_ref, out_ref, *, dh):
    m = ht_ref.shape[0]
    d128 = _disp128(disp_ref, m, dh)
    S = (s_ref[0] + s_ref[1])[:m]
    h3 = d128 * (S + ht_ref[...]) + b_ref[...][None, :]
    halves = []
    for k in (0, 1):
        hh = h3[:, k * dh:(k + 1) * dh]
        mx = jnp.max(hh, axis=-1, keepdims=True)
        ex = jnp.exp(hh - mx)
        halves.append(ex / jnp.sum(ex, axis=-1, keepdims=True))
    out_ref[...] = jnp.concatenate(halves, axis=1)


def _tc1(disp, xp, wbd, dh):
    m = xp.shape[0]
    return pl.pallas_call(
        functools.partial(_tc1_body, dh=dh),
        out_shape=jax.ShapeDtypeStruct((m, 2 * dh), _f32),
    )(disp, xp, wbd)


def _tc2(sp, htp, disp, bp, wbd, dh):
    m = htp.shape[0]
    return pl.pallas_call(
        functools.partial(_tc2_body, dh=dh),
        out_shape=(jax.ShapeDtypeStruct((m, 2 * dh), _f32),
                   jax.ShapeDtypeStruct((m, 2 * dh), _f32)),
    )(sp, htp, disp, bp, wbd)


def _tc3(sp, htp, disp, bp, h1p, wabd, wbbd, dh):
    m = htp.shape[0]
    return pl.pallas_call(
        functools.partial(_tc3_body, dh=dh),
        out_shape=jax.ShapeDtypeStruct((m, 2 * dh), _f32),
    )(sp, htp, disp, bp, h1p, wabd, wbbd)


def _tc4(sp, htp, disp, bp, dh):
    m = htp.shape[0]
    return pl.pallas_call(
        functools.partial(_tc4_body, dh=dh),
        out_shape=jax.ShapeDtypeStruct((m, 2 * dh), _f32),
    )(sp, htp, disp, bp)


def kernel(x, edge_index, edge_attr, W1, b1, W2, b2, W3, b3):
    n = x.shape[0]
    e = edge_index.shape[1]
    assert n <= NS * RT

    quantum = NW * NBUF * GROUP
    e2 = ((e + quantum - 1) // quantum) * quantum
    idx = edge_index.astype(_i32)
    if e2 != e:
        spread = jnp.arange(e2 - e, dtype=_i32) % jnp.int32(n)
        idx = jnp.concatenate([idx, jnp.stack([spread, spread])], axis=1)
        ewf = jnp.concatenate([edge_attr, jnp.zeros((e2 - e,), _f32)])
    else:
        ewf = edge_attr
    pk2 = (idx[0] | (idx[1] << 16)).reshape(e2 // GROUP, GROUP)
    ew2 = ewf.reshape(e2 // GROUP, GROUP)

    dh = W1.shape[1]
    assert n % 2 == 0 and 2 * dh == 128
    nh = n // 2
    sr = NS * RT // 2

    def _bd(w):
        z = jnp.zeros_like(w)
        return jnp.block([[w, z], [z, w]])

    xp = x.reshape(nh, 2 * x.shape[1])
    w1bd, w2bd = _bd(W1), _bd(W2)
    w3abd, w3bbd = _bd(W3[:dh]), _bd(W3[dh:])
    b1p = jnp.concatenate([b1, b1])
    b2p = jnp.concatenate([b2, b2])
    b3p = jnp.concatenate([b3, b3])

    degp = _deg_call(pk2, ew2)
    disp = lax.rsqrt(degp[0] + degp[1] + 1.0).reshape(sr, 2)

    ht1p = _tc1(disp, xp, w1bd, dh)
    s1 = _spmv_call(ht1p.reshape(n, dh), pk2, ewf)
    h1p, ht2p = _tc2(s1.reshape(NC, sr, 2 * dh), ht1p, disp, b1p, w2bd, dh)
    s2 = _spmv_call(ht2p.reshape(n, dh), pk2, ewf)
    ht3p = _tc3(s2.reshape(NC, sr, 2 * dh), ht2p, disp, b2p, h1p,
                w3abd, w3bbd, dh)
    s3 = _spmv_call(ht3p.reshape(n, dh), pk2, ewf)
    outp = _tc4(s3.reshape(NC, sr, 2 * dh), ht3p, disp, b3p, dh)
    return outp.reshape(n, dh)


# speedup vs baseline: 1.1204x; 1.1204x over previous
import functools

import jax
import jax.numpy as jnp
from jax import lax
from jax.experimental import pallas as pl
from jax.experimental.pallas import tpu as pltpu
from jax.experimental.pallas import tpu_sc as plsc

NC, NS, L = 2, 16, 16
NW = NC * NS
GROUP = 128
RT = 640

_f32 = jnp.float32
_i32 = jnp.int32


def _full16(v):
    return jnp.full((L,), v, _i32)


def _zeros16():
    return jnp.zeros((L,), _f32)


def _deg_body(idx3, ew2, degp, dstv, ewv, zbuf, acc, *, rpw):
    c = lax.axis_index("c")
    s = lax.axis_index("s")
    wid = s * NC + c

    def _zb(i, _):
        zbuf[pl.ds(i * L, L)] = _zeros16()
        return 0
    lax.fori_loop(0, RT // L, _zb, 0)
    pltpu.sync_copy(zbuf, acc.at[pl.ds(s * RT, RT)])
    plsc.subcore_barrier()

    pltpu.sync_copy(idx3.at[1, pl.ds(wid * rpw, rpw)], dstv)
    pltpu.sync_copy(ew2.at[pl.ds(wid * rpw, rpw)], ewv)

    def _row(j, _):
        pltpu.sync_copy(ewv.at[j], acc.at[dstv.at[j]], add=True)
        return 0
    lax.fori_loop(0, rpw, _row, 0)

    plsc.subcore_barrier()
    pltpu.sync_copy(acc.at[pl.ds(s * RT, RT)], degp.at[c, pl.ds(s * RT, RT)])


NBUF = 5


def _spmv_body(ht, idx3, ewf, out, *args, rpw, d):
    srcv, dstv, ewv = args[0], args[1], args[2]
    rows = args[3:3 + NBUF]
    gsems, ssems, acc = args[3 + NBUF], args[4 + NBUF], args[5 + NBUF]
    c = lax.axis_index("c")
    s = lax.axis_index("s")
    wid = s * NC + c

    pltpu.async_copy(idx3.at[0, pl.ds(wid * rpw, rpw)], srcv, gsems.at[0])
    pltpu.async_copy(idx3.at[1, pl.ds(wid * rpw, rpw)], dstv, gsems.at[1])
    pltpu.async_copy(ewf.at[pl.ds(wid * rpw * GROUP, rpw * GROUP)],
                     ewv.at[pl.ds(0, rpw * GROUP)], gsems.at[2])

    def _zr(e, _):
        for k in range(d // L):
            rows[0][e, pl.ds(k * L, L)] = _zeros16()
        return 0
    lax.fori_loop(0, rows[0].shape[0], _zr, 0)
    for k in range(RT // rows[0].shape[0]):
        pltpu.sync_copy(rows[0], acc.at[pl.ds(s * RT + k * rows[0].shape[0],
                                              rows[0].shape[0])])

    pltpu.make_async_copy(idx3.at[0, pl.ds(wid * rpw, rpw)], srcv,
                          gsems.at[0]).wait()
    pltpu.make_async_copy(idx3.at[1, pl.ds(wid * rpw, rpw)], dstv,
                          gsems.at[1]).wait()
    pltpu.make_async_copy(ewf.at[pl.ds(wid * rpw * GROUP, rpw * GROUP)],
                          ewv.at[pl.ds(0, rpw * GROUP)], gsems.at[2]).wait()

    plsc.subcore_barrier()

    def _gather(j, b):
        pltpu.async_copy(ht.at[srcv.at[j]], rows[b].at[pl.ds(0, GROUP)],
                         gsems.at[b])

    def _gwait(j, b):
        pltpu.make_async_copy(ht.at[srcv.at[j]], rows[b].at[pl.ds(0, GROUP)],
                              gsems.at[b]).wait()

    def _mul(j, b):
        base = j * GROUP
        buf = rows[b]

        def mbody(g):
            e0 = g * L
            w16 = ewv[pl.ds(base + e0, L)]
            for t in range(L):
                w = w16[jnp.full((L,), t, _i32)]
                for k in range(d // L):
                    sl = pl.ds(k * L, L)
                    buf[e0 + t, sl] = buf[e0 + t, sl] * w
        plsc.parallel_loop(0, GROUP // L, 1, unroll=4)(mbody)

    for b in range(NBUF):
        _gather(b, b)

    def _quad(i, _):
        j = i * NBUF
        descs = []
        for b in range(NBUF):
            _gwait(j + b, b)
            _mul(j + b, b)
            descs.append(pltpu.async_copy(rows[b].at[pl.ds(0, GROUP)],
                                          acc.at[dstv.at[j + b]], ssems.at[b],
                                          add=True))
        for b in range(NBUF):
            descs[b].wait()

            @pl.when(j + b + NBUF < rpw)
            def _():
                _gather(j + b + NBUF, b)
        return 0
    lax.fori_loop(0, rpw // NBUF, _quad, 0)

    plsc.subcore_barrier()
    pltpu.sync_copy(acc.at[pl.ds(s * RT, RT)], out.at[c, pl.ds(s * RT, RT), :])


def _deg_call(idx3, ew2):
    _, rows2, g = idx3.shape
    rpw = rows2 // NW
    mesh = plsc.VectorSubcoreMesh(core_axis_name="c", subcore_axis_name="s")
    body = functools.partial(_deg_body, rpw=rpw)
    return pl.kernel(
        body,
        out_type=jax.ShapeDtypeStruct((NC, NS * RT), _f32),
        mesh=mesh,
        scratch_types=[
            pltpu.VMEM((rpw, g), _i32),
            pltpu.VMEM((rpw, g), _f32),
            pltpu.VMEM((RT,), _f32),
            pltpu.VMEM_SHARED((NS * RT,), _f32),
        ],
    )(idx3, ew2)


def _spmv_call(ht, idx3, ewf):
    d = ht.shape[1]
    _, rows2, g = idx3.shape
    rpw = rows2 // NW
    mesh = plsc.VectorSubcoreMesh(core_axis_name="c", subcore_axis_name="s")
    body = functools.partial(_spmv_body, rpw=rpw, d=d)
    return pl.kernel(
        body,
        out_type=jax.ShapeDtypeStruct((NC, NS * RT, d), _f32),
        mesh=mesh,
        compiler_params=pltpu.CompilerParams(use_tc_tiling_on_sc=False),
        scratch_types=[
            pltpu.VMEM((rpw, g), _i32),
            pltpu.VMEM((rpw, g), _i32),
            pltpu.VMEM((rpw * g + L,), _f32),
        ] + [pltpu.VMEM((128, d), _f32) for _ in range(NBUF)] + [
            pltpu.SemaphoreType.DMA((NBUF,)),
            pltpu.SemaphoreType.DMA((NBUF,)),
            pltpu.VMEM_SHARED((NS * RT, d), _f32),
        ],
    )(ht, idx3, ewf)


def _disp128(disp_ref, m, dh):
    de = disp_ref[0:m, 0:1]
    do = disp_ref[0:m, 1:2]
    return jnp.concatenate([jnp.broadcast_to(de, (m, dh)),
                            jnp.broadcast_to(do, (m, dh))], axis=1)


def _tc1_body(disp_ref, x_ref, w_ref, ht_ref, *, dh):
    m = x_ref.shape[0]
    d128 = _disp128(disp_ref, m, dh)
    ht_ref[...] = jnp.dot(x_ref[...], w_ref[...],
                          preferred_element_type=_f32) * d128


def _tc2_body(s_ref, ht_ref, disp_ref, b_ref, w_ref, h_ref, htn_ref, *, dh):
    m = ht_ref.shape[0]
    d128 = _disp128(disp_ref, m, dh)
    S = (s_ref[0] + s_ref[1])[:m]
    h = d128 * (S + ht_ref[...]) + b_ref[...][None, :]
    h_ref[...] = h
    htn_ref[...] = jnp.dot(h, w_ref[...], preferred_element_type=_f32) * d128


def _tc3_body(s_ref, ht_ref, disp_ref, b_ref, h1_ref, wa_ref, wb_ref,
              htn_ref, *, dh):
    m = ht_ref.shape[0]
    d128 = _disp128(disp_ref, m, dh)
    S = (s_ref[0] + s_ref[1])[:m]
    h2 = d128 * (S + ht_ref[...]) + b_ref[...][None, :]
    acc = (jnp.dot(h1_ref[...], wa_ref[...], preferred_element_type=_f32)
           + jnp.dot(h2, wb_ref[...], preferred_element_type=_f32))
    htn_ref[...] = acc * d128


def _tc4_body(s_ref, ht_ref, disp_ref, b_ref, out_ref, *, dh):
    m = ht_ref.shape[0]
    d128 = _disp128(disp_ref, m, dh)
    S = (s_ref[0] + s_ref[1])[:m]
    h3 = d128 * (S + ht_ref[...]) + b_ref[...][None, :]
    halves = []
    for k in (0, 1):
        hh = h3[:, k * dh:(k + 1) * dh]
        mx = jnp.max(hh, axis=-1, keepdims=True)
        ex = jnp.exp(hh - mx)
        halves.append(ex / jnp.sum(ex, axis=-1, keepdims=True))
    out_ref[...] = jnp.concatenate(halves, axis=1)


def _tc1(disp, xp, wbd, dh):
    m = xp.shape[0]
    return pl.pallas_call(
        functools.partial(_tc1_body, dh=dh),
        out_shape=jax.ShapeDtypeStruct((m, 2 * dh), _f32),
    )(disp, xp, wbd)


def _tc2(sp, htp, disp, bp, wbd, dh):
    m = htp.shape[0]
    return pl.pallas_call(
        functools.partial(_tc2_body, dh=dh),
        out_shape=(jax.ShapeDtypeStruct((m, 2 * dh), _f32),
                   jax.ShapeDtypeStruct((m, 2 * dh), _f32)),
    )(sp, htp, disp, bp, wbd)


def _tc3(sp, htp, disp, bp, h1p, wabd, wbbd, dh):
    m = htp.shape[0]
    return pl.pallas_call(
        functools.partial(_tc3_body, dh=dh),
        out_shape=jax.ShapeDtypeStruct((m, 2 * dh), _f32),
    )(sp, htp, disp, bp, h1p, wabd, wbbd)


def _tc4(sp, htp, disp, bp, dh):
    m = htp.shape[0]
    return pl.pallas_call(
        functools.partial(_tc4_body, dh=dh),
        out_shape=jax.ShapeDtypeStruct((m, 2 * dh), _f32),
    )(sp, htp, disp, bp)


def kernel(x, edge_index, edge_attr, W1, b1, W2, b2, W3, b3):
    n = x.shape[0]
    e = edge_index.shape[1]
    assert n <= NS * RT

    quantum = NW * NBUF * GROUP
    e2 = ((e + quantum - 1) // quantum) * quantum
    idx = edge_index.astype(_i32)
    if e2 != e:
        spread = jnp.arange(e2 - e, dtype=_i32) % jnp.int32(n)
        idx = jnp.concatenate([idx, jnp.stack([spread, spread])], axis=1)
        ewf = jnp.concatenate([edge_attr, jnp.zeros((e2 - e,), _f32)])
    else:
        ewf = edge_attr
    idx3 = idx.reshape(2, e2 // GROUP, GROUP)
    ew2 = ewf.reshape(e2 // GROUP, GROUP)

    dh = W1.shape[1]
    assert n % 2 == 0 and 2 * dh == 128
    nh = n // 2
    sr = NS * RT // 2

    def _bd(w):
        z = jnp.zeros_like(w)
        return jnp.block([[w, z], [z, w]])

    xp = x.reshape(nh, 2 * x.shape[1])
    w1bd, w2bd = _bd(W1), _bd(W2)
    w3abd, w3bbd = _bd(W3[:dh]), _bd(W3[dh:])
    b1p = jnp.concatenate([b1, b1])
    b2p = jnp.concatenate([b2, b2])
    b3p = jnp.concatenate([b3, b3])

    degp = _deg_call(idx3, ew2)
    disp = lax.rsqrt(degp[0] + degp[1] + 1.0).reshape(sr, 2)

    ht1p = _tc1(disp, xp, w1bd, dh)
    s1 = _spmv_call(ht1p.reshape(n, dh), idx3, ewf)
    h1p, ht2p = _tc2(s1.reshape(NC, sr, 2 * dh), ht1p, disp, b1p, w2bd, dh)
    s2 = _spmv_call(ht2p.reshape(n, dh), idx3, ewf)
    ht3p = _tc3(s2.reshape(NC, sr, 2 * dh), ht2p, disp, b2p, h1p,
                w3abd, w3bbd, dh)
    s3 = _spmv_call(ht3p.reshape(n, dh), idx3, ewf)
    outp = _tc4(s3.reshape(NC, sr, 2 * dh), ht3p, disp, b3p, dh)
    return outp.reshape(n, dh)
